# Initial kernel scaffold; baseline (speedup 1.0000x reference)
import jax
import jax.numpy as jnp
from jax.experimental import pallas as pl


def kernel(patch_x, patch_edge_index, patch_edge_weight, patch_node_graph_ids, mesh_edge_index, pW1, pW2, pGN1_g, pGN1_b, pGN1_a, pGN2_g, pGN2_b, pGN2_a, p_lin_W, p_cls_W, GNpr_g, GNpr_b, GNpr_a, mW1, mW2, mGN1_g, mGN1_b, mGN1_a, mGN2_g, mGN2_b, mGN2_a, m_lin_W, m_lin_b, m_cls_W):
    raise NotImplementedError("write your pallas kernel here")


# speedup vs baseline: 5.8875x; 5.8875x over previous
import functools

import jax
import jax.numpy as jnp
from jax import lax
from jax.experimental import pallas as pl
from jax.experimental.pallas import tpu as pltpu
from jax.experimental.pallas import tpu_sc as plsc

N_NODES = 10000
E_PATCH = 320000
P = 500
E_MESH = 8000
D_IN = 128
H_INT = 128
H_HALF = 64
R = 64
H_MESH = 128
OUT = 16
EPS = 1e-5
SLOPE = 0.01

NC = 2
NS = 16
NW = NC * NS

SB = 32
JC = 8
KE = SB * JC
NSB = E_PATCH // SB
NCHUNK = NSB // JC
NPAD = 10240


def _leaky(x):
    return jnp.where(x >= 0, x, SLOPE * x)


def _graph_norm(x, g, b, a):
    mean = jnp.mean(x, axis=0, keepdims=True)
    o = x - a * mean
    var = jnp.mean(o * o, axis=0, keepdims=True)
    return g * o / jnp.sqrt(var + EPS) + b


def _inst_norm(x):
    m = jnp.mean(x, axis=-1, keepdims=True)
    v = jnp.mean((x - m) ** 2, axis=-1, keepdims=True)
    return (x - m) / jnp.sqrt(v + EPS)


ZB = 2048


@functools.partial(
    pl.kernel,
    out_type=jax.ShapeDtypeStruct((NC, 2, NPAD), jnp.float32),
    mesh=plsc.VectorSubcoreMesh(core_axis_name="c", subcore_axis_name="s"),
    scratch_types=[
        pltpu.VMEM((JC, SB), jnp.int32),
        pltpu.VMEM((ZB,), jnp.float32),
        pltpu.VMEM((SB,), jnp.float32),
        pltpu.VMEM_SHARED((NPAD,), jnp.float32),
        pltpu.VMEM_SHARED((NPAD,), jnp.float32),
        pltpu.SemaphoreType.DMA,
    ],
)
def _sc_degrees(src_hbm, dst_hbm, out_hbm, idx_v, zv, ones_v, dsh_o, dsh_i,
                sem):
    cid = lax.axis_index("c")
    sid = lax.axis_index("s")
    wid = sid * NC + cid
    z16 = jnp.zeros((16,), jnp.float32)
    o16 = jnp.ones((16,), jnp.float32)

    def zbody(i, _):
        zv[pl.ds(i * 16, 16)] = z16
        return 0

    lax.fori_loop(0, ZB // 16, zbody, 0)
    for i in range(SB // 16):
        ones_v[pl.ds(i * 16, 16)] = o16

    nslab = NPAD // ZB

    @pl.when(sid < nslab)
    def _():
        pltpu.sync_copy(zv, dsh_o.at[pl.ds(sid * ZB, ZB)])

    @pl.when(jnp.logical_and(sid >= nslab, sid < 2 * nslab))
    def _():
        pltpu.sync_copy(zv, dsh_i.at[pl.ds((sid - nslab) * ZB, ZB)])

    plsc.subcore_barrier()

    nloc = (NCHUNK - wid + NW - 1) // NW

    def scatter_chunks(eidx_hbm, dsh):
        def body(i, _):
            sb0 = pl.multiple_of((wid + i * NW) * JC, JC)
            pltpu.sync_copy(eidx_hbm.at[pl.ds(sb0, JC)], idx_v)
            cps = [
                pltpu.async_copy(ones_v, dsh.at[idx_v.at[j]], sem, add=True)
                for j in range(JC)
            ]
            for c in cps:
                c.wait()
            return 0

        lax.fori_loop(0, nloc, body, 0)

    scatter_chunks(src_hbm, dsh_o)
    scatter_chunks(dst_hbm, dsh_i)
    plsc.subcore_barrier()

    spt = NPAD // NS
    pltpu.sync_copy(dsh_o.at[pl.ds(sid * spt, spt)],
                    out_hbm.at[cid, 0, pl.ds(sid * spt, spt)])
    pltpu.sync_copy(dsh_i.at[pl.ds(sid * spt, spt)],
                    out_hbm.at[cid, 1, pl.ds(sid * spt, spt)])


def _make_sc_conv(W):
    @functools.partial(
        pl.kernel,
        out_type=jax.ShapeDtypeStruct((NC, NPAD, W), jnp.float32),
        mesh=plsc.VectorSubcoreMesh(core_axis_name="c", subcore_axis_name="s"),
        scratch_types=[
            pltpu.VMEM((JC, SB), jnp.int32),
            pltpu.VMEM((JC, SB), jnp.int32),
            pltpu.VMEM((KE,), jnp.float32),
            pltpu.VMEM((KE, W), jnp.float32),
            pltpu.VMEM_SHARED((NPAD, W), jnp.float32),
            pltpu.SemaphoreType.DMA,
        ],
    )
    def conv(src_hbm, dst_hbm, ew_hbm, y_hbm, out_hbm,
             srci, dsti, ewv, rows, agg_sh, sem):
        cid = lax.axis_index("c")
        sid = lax.axis_index("s")
        wid = sid * NC + cid
        z16 = jnp.zeros((16,), jnp.float32)

        def zrows(i, _):
            for c2 in range(W // 16):
                rows[i, pl.ds(c2 * 16, 16)] = z16
            return 0

        lax.fori_loop(0, KE, zrows, 0)

        nzc = NPAD // KE
        for j in range((nzc + NS - 1) // NS):
            zc = sid + j * NS
            zoff = pl.multiple_of(zc * KE, KE)
            if (j + 1) * NS <= nzc:
                pltpu.sync_copy(rows, agg_sh.at[pl.ds(zoff, KE)])
            else:
                @pl.when(zc < nzc)
                def _():
                    pltpu.sync_copy(rows, agg_sh.at[pl.ds(zoff, KE)])

        plsc.subcore_barrier()

        nloc = (NCHUNK - wid + NW - 1) // NW

        def body(i, _):
            cb = wid + i * NW
            sb0 = pl.multiple_of(cb * JC, JC)
            eb0 = pl.multiple_of(cb * KE, KE)
            pltpu.sync_copy(src_hbm.at[pl.ds(sb0, JC)], srci)
            pltpu.sync_copy(dst_hbm.at[pl.ds(sb0, JC)], dsti)
            pltpu.sync_copy(ew_hbm.at[pl.ds(eb0, KE)], ewv)
            cps = [
                pltpu.async_copy(
                    y_hbm.at[srci.at[j]], rows.at[pl.ds(j * SB, SB)], sem)
                for j in range(JC)
            ]
            for c in cps:
                c.wait()

            def escale(g, _):
                ew16 = ewv[pl.ds(g * 16, 16)]
                for r in range(16):
                    e = g * 16 + r
                    bw = jnp.full((16,), ew16[r], jnp.float32)
                    for c2 in range(W // 16):
                        sl = pl.ds(c2 * 16, 16)
                        rows[e, sl] = rows[e, sl] * bw
                return 0

            lax.fori_loop(0, KE // 16, escale, 0)
            for j in range(JC):
                pltpu.sync_copy(
                    rows.at[pl.ds(j * SB, SB)], agg_sh.at[dsti.at[j]],
                    add=True)
            return 0

        lax.fori_loop(0, nloc, body, 0)
        plsc.subcore_barrier()

        rpt = NPAD // NS
        pltpu.sync_copy(
            agg_sh.at[pl.ds(sid * rpt, rpt)],
            out_hbm.at[cid, pl.ds(sid * rpt, rpt)])

    return conv


_sc_conv128 = _make_sc_conv(H_INT)


def _tc_pre_body(x_ref, w_ref, degp_ref, y_ref, dini_ref, douti_ref):
    deg = (degp_ref[0] + degp_ref[1])[:, :N_NODES]
    douti = lax.rsqrt(jnp.maximum(deg[0], 1.0))
    douti_ref[...] = douti[:, None]
    dini_ref[...] = lax.rsqrt(jnp.maximum(deg[1], 1.0))[:, None]
    y = jnp.dot(x_ref[...], w_ref[...], preferred_element_type=jnp.float32)
    y_ref[...] = y * douti[:, None]


def _tc_pre(x, w, degp):
    return pl.pallas_call(
        _tc_pre_body,
        out_shape=(
            jax.ShapeDtypeStruct((N_NODES, H_INT), jnp.float32),
            jax.ShapeDtypeStruct((N_NODES, 1), jnp.float32),
            jax.ShapeDtypeStruct((N_NODES, 1), jnp.float32),
        ),
    )(x, w, degp)


def _tc_mid_body(part_ref, dini_ref, douti_ref, g_ref, b_ref, a_ref, w2_ref,
                 y2_ref):
    h = (part_ref[0][:N_NODES] + part_ref[1][:N_NODES]) * dini_ref[...]
    h = _leaky(h)
    h = _graph_norm(h, g_ref[...], b_ref[...], a_ref[...])
    y2 = jnp.dot(h, w2_ref[...], preferred_element_type=jnp.float32)
    y2_ref[...] = y2 * douti_ref[...]


def _tc_mid(part, dini, douti, g, b, a, w2):
    return pl.pallas_call(
        _tc_mid_body,
        out_shape=jax.ShapeDtypeStruct((N_NODES, H_INT), jnp.float32),
    )(part, dini, douti, g, b, a, w2)


def _tc_final_body(part_ref, dini_ref, g2_ref, b2_ref, a2_ref, ids_ref,
                   msrc_ref, mdst_ref, plin_ref, pcls_ref,
                   gpr_g_ref, gpr_b_ref, gpr_a_ref,
                   mW1_ref, mW2_ref, m1g_ref, m1b_ref, m1a_ref,
                   m2g_ref, m2b_ref, m2a_ref, mlinW_ref, mlinb_ref,
                   mclsW_ref, mesh_out_ref, readouts_ref):
    h = (part_ref[0][:N_NODES, :H_HALF] + part_ref[1][:N_NODES, :H_HALF])
    h = h * dini_ref[...]
    h = _leaky(h)
    h = _graph_norm(h, g2_ref[...], b2_ref[...], a2_ref[...])

    ids = ids_ref[...]
    piota = lax.broadcasted_iota(jnp.int32, (1, P), 1)
    sums = jnp.zeros((P, H_HALF), jnp.float32)
    counts = jnp.zeros((P,), jnp.float32)
    NB = 1000
    for nb in range(N_NODES // NB):
        oh = (ids[nb * NB:(nb + 1) * NB][:, None] == piota)
        oh = oh.astype(jnp.float32)
        hb = h[nb * NB:(nb + 1) * NB]
        sums = sums + lax.dot_general(
            oh, hb, (((0,), (0,)), ((), ())),
            preferred_element_type=jnp.float32)
        counts = counts + jnp.sum(oh, axis=0)
    r = sums / jnp.maximum(counts, 1.0)[:, None]

    r = _leaky(jnp.dot(r, plin_ref[...], preferred_element_type=jnp.float32))
    r = _inst_norm(r)
    ro = jnp.dot(r, pcls_ref[...], preferred_element_type=jnp.float32)
    ro = _leaky(ro)
    ro = _graph_norm(ro, gpr_g_ref[...], gpr_b_ref[...], gpr_a_ref[...])
    readouts_ref[...] = ro

    msrc = msrc_ref[...]
    mdst = mdst_ref[...]
    A = jnp.zeros((P, P), jnp.float32)
    EB = 1000
    for eb in range(E_MESH // EB):
        ohd = (mdst[eb * EB:(eb + 1) * EB][:, None] == piota)
        ohs = (msrc[eb * EB:(eb + 1) * EB][:, None] == piota)
        A = A + lax.dot_general(
            ohd.astype(jnp.float32), ohs.astype(jnp.float32),
            (((0,), (0,)), ((), ())), preferred_element_type=jnp.float32)
    mdouti = lax.rsqrt(jnp.maximum(jnp.sum(A, axis=0), 1.0))[:, None]
    mdini = lax.rsqrt(jnp.maximum(jnp.sum(A, axis=1), 1.0))[:, None]

    def mesh_conv(x, wref):
        t = jnp.dot(x * mdouti, wref[...], preferred_element_type=jnp.float32)
        agg = jnp.dot(A, t, preferred_element_type=jnp.float32)
        return agg * mdini

    u = mesh_conv(ro, mW1_ref)
    u = _leaky(u)
    u = _graph_norm(u, m1g_ref[...], m1b_ref[...], m1a_ref[...])
    u = mesh_conv(u, mW2_ref)
    u = _leaky(u)
    u = _graph_norm(u, m2g_ref[...], m2b_ref[...], m2a_ref[...])

    pooled = jnp.mean(u, axis=0, keepdims=True)
    pooled = jnp.dot(pooled, mlinW_ref[...],
                     preferred_element_type=jnp.float32) + mlinb_ref[...]
    pooled = _leaky(pooled)
    pooled = _inst_norm(pooled)
    mesh_out_ref[...] = jnp.dot(pooled, mclsW_ref[...],
                                preferred_element_type=jnp.float32)


def _tc_final(part, dini, g2, b2, a2, ids, msrc, mdst, plinW, pclsW,
              gpr_g, gpr_b, gpr_a, mW1, mW2, m1g, m1b, m1a, m2g, m2b, m2a,
              mlinW, mlinb, mclsW):
    return pl.pallas_call(
        _tc_final_body,
        out_shape=(
            jax.ShapeDtypeStruct((1, OUT), jnp.float32),
            jax.ShapeDtypeStruct((P, R), jnp.float32),
        ),
    )(part, dini, g2, b2, a2, ids, msrc, mdst, plinW, pclsW,
      gpr_g, gpr_b, gpr_a, mW1, mW2, m1g, m1b, m1a, m2g, m2b, m2a,
      mlinW, mlinb, mclsW)


def kernel(patch_x, patch_edge_index, patch_edge_weight, patch_node_graph_ids,
           mesh_edge_index, pW1, pW2, pGN1_g, pGN1_b, pGN1_a, pGN2_g, pGN2_b,
           pGN2_a, p_lin_W, p_cls_W, GNpr_g, GNpr_b, GNpr_a, mW1, mW2, mGN1_g,
           mGN1_b, mGN1_a, mGN2_g, mGN2_b, mGN2_a, m_lin_W, m_lin_b, m_cls_W):
    src = patch_edge_index[0]
    dst = patch_edge_index[1]
    src2d = src.reshape(NSB, SB)
    dst2d = dst.reshape(NSB, SB)

    degp = _sc_degrees(src2d, dst2d)
    y1, dini, douti = _tc_pre(patch_x, pW1, degp)
    part1 = _sc_conv128(src2d, dst2d, patch_edge_weight, y1)
    pW2p = jnp.pad(pW2, ((0, 0), (0, H_INT - H_HALF)))
    y2 = _tc_mid(part1, dini, douti, pGN1_g, pGN1_b, pGN1_a, pW2p)
    part2 = _sc_conv128(src2d, dst2d, patch_edge_weight, y2)
    mesh_out, readouts = _tc_final(
        part2, dini, pGN2_g, pGN2_b, pGN2_a, patch_node_graph_ids,
        mesh_edge_index[0], mesh_edge_index[1], p_lin_W, p_cls_W,
        GNpr_g, GNpr_b, GNpr_a, mW1, mW2, mGN1_g, mGN1_b, mGN1_a,
        mGN2_g, mGN2_b, mGN2_a, m_lin_W, m_lin_b, m_cls_W)
    return (mesh_out, readouts)
